# baseline (speedup 1.0000x reference)
import jax
import jax.numpy as jnp
from jax.experimental import pallas as pl
from jax.experimental.pallas import tpu as pltpu

_NUM_EMBEDDINGS = 1024
_EMBEDDING_DIM = 64
_COMMITMENT_COST = 0.25
_SPATIAL = 32 * 32
_BATCH = 16
_IMGS_PER_STEP = 8
_GRID = _BATCH // _IMGS_PER_STEP
_N_TOTAL = _BATCH * _SPATIAL * _EMBEDDING_DIM


def _vq_body(x_ref, w_ref, q_ref, idx_ref, loss_ref, esum_ref, iota_ref):
    w = w_ref[...]

    @pl.when(pl.program_id(0) == 0)
    def _():
        iota_ref[...] = jax.lax.broadcasted_iota(
            jnp.int32, (_NUM_EMBEDDINGS, _SPATIAL), 0).astype(jnp.float32)
        esum_ref[...] = jnp.zeros((1, 256), jnp.float32)
        loss_ref[...] = jnp.zeros((1, 1), jnp.float32)

    wsq = jnp.sum(w * w, axis=1, keepdims=True)
    w2 = w + w
    code_iota_f = iota_ref[...]

    def one_image(c, acc):
        x = x_ref[c].reshape(_EMBEDDING_DIM, _SPATIAL)

        xsq = jnp.sum(x * x, axis=0, keepdims=True)
        xw2 = jax.lax.dot_general(
            w2, x, (((1,), (0,)), ((), ())),
            preferred_element_type=jnp.float32)
        dist = (xsq + wsq) - xw2

        idx_f = jnp.argmin(dist, axis=0).astype(jnp.float32)[None]

        onehot = jnp.where(code_iota_f == idx_f, 1.0, 0.0)
        q = jax.lax.dot_general(
            w, onehot, (((0,), (0,)), ((), ())),
            preferred_element_type=jnp.float32)

        q_ref[c] = x + (q - x)
        idx_ref[c, 0, :] = idx_f[0].astype(jnp.int32)

        diff = q - x
        return acc + jnp.sum(diff * diff, axis=(0, 1), keepdims=True)

    acc = jax.lax.fori_loop(
        0, _IMGS_PER_STEP, one_image, jnp.zeros((1, 1), jnp.float32),
        unroll=True)
    total = loss_ref[...] + acc
    scaled = _COMMITMENT_COST * (total / _N_TOTAL) * 10.0
    loss_ref[...] = jnp.where(
        pl.program_id(0) == _GRID - 1, scaled, total)


def kernel(inputs, W):
    x = inputs.astype(jnp.float32)
    q, idx, loss, esum = pl.pallas_call(
        _vq_body,
        grid=(_GRID,),
        in_specs=[
            pl.BlockSpec((_IMGS_PER_STEP, _EMBEDDING_DIM, 32, 32),
                         lambda i: (i, 0, 0, 0)),
            pl.BlockSpec((_NUM_EMBEDDINGS, _EMBEDDING_DIM), lambda i: (0, 0)),
        ],
        out_specs=[
            pl.BlockSpec((_IMGS_PER_STEP, _EMBEDDING_DIM, _SPATIAL),
                         lambda i: (i, 0, 0)),
            pl.BlockSpec((_IMGS_PER_STEP, 1, _SPATIAL), lambda i: (i, 0, 0)),
            pl.BlockSpec((1, 1), lambda i: (0, 0)),
            pl.BlockSpec((1, 256), lambda i: (0, 0)),
        ],
        out_shape=[
            jax.ShapeDtypeStruct((_BATCH, _EMBEDDING_DIM, _SPATIAL), jnp.float32),
            jax.ShapeDtypeStruct((_BATCH, 1, _SPATIAL), jnp.int32),
            jax.ShapeDtypeStruct((1, 1), jnp.float32),
            jax.ShapeDtypeStruct((1, 256), jnp.float32),
        ],
        scratch_shapes=[pltpu.VMEM((_NUM_EMBEDDINGS, _SPATIAL), jnp.float32)],
        compiler_params=pltpu.CompilerParams(
            dimension_semantics=("arbitrary",),
        ),
    )(x, W)

    quantized_st = q.reshape(_BATCH, _EMBEDDING_DIM, 32, 32)
    encoding_indices = idx.reshape(_BATCH, 32, 32)
    return (quantized_st, loss[0, 0], encoding_indices, esum[0], W)


# speedup vs baseline: 1.3532x; 1.3532x over previous
import jax
import jax.numpy as jnp
from jax.experimental import pallas as pl
from jax.experimental.pallas import tpu as pltpu

_NUM_EMBEDDINGS = 1024
_EMBEDDING_DIM = 64
_COMMITMENT_COST = 0.25
_SPATIAL = 32 * 32
_BATCH = 16
_IMGS_PER_STEP = 8
_GRID = _BATCH // _IMGS_PER_STEP
_N_TOTAL = _BATCH * _SPATIAL * _EMBEDDING_DIM


def _vq_body(x_ref, w_ref, q_ref, idx_ref, loss_ref, esum_ref):
    w = w_ref[...]

    @pl.when(pl.program_id(0) == 0)
    def _():
        esum_ref[...] = jnp.zeros((1, 256), jnp.float32)
        loss_ref[...] = jnp.zeros((1, 1), jnp.float32)

    wsq = jnp.sum(w * w, axis=1, keepdims=True)
    w2 = w + w
    code_iota = jax.lax.broadcasted_iota(
        jnp.int32, (_NUM_EMBEDDINGS, _SPATIAL), 0)

    def one_image(c, acc):
        x = x_ref[c]

        xsq = jnp.sum(x * x, axis=0, keepdims=True)
        xw2 = jax.lax.dot_general(
            w2, x, (((1,), (0,)), ((), ())),
            preferred_element_type=jnp.float32)
        dist = (xsq + wsq) - xw2

        idx = jnp.argmin(dist, axis=0)[None]

        onehot = jnp.where(code_iota == idx, 1.0, 0.0)
        q = jax.lax.dot_general(
            w, onehot, (((0,), (0,)), ((), ())),
            preferred_element_type=jnp.float32)

        q_ref[c] = x + (q - x)
        idx_ref[c, 0, :] = idx[0]

        diff = q - x
        return acc + jnp.sum(diff * diff, axis=(0, 1), keepdims=True)

    acc = jax.lax.fori_loop(
        0, _IMGS_PER_STEP, one_image, jnp.zeros((1, 1), jnp.float32),
        unroll=True)
    total = loss_ref[...] + acc
    scaled = _COMMITMENT_COST * (total / _N_TOTAL) * 10.0
    loss_ref[...] = jnp.where(
        pl.program_id(0) == _GRID - 1, scaled, total)


def kernel(inputs, W):
    x = inputs.astype(jnp.float32).reshape(_BATCH, _EMBEDDING_DIM, _SPATIAL)
    q, idx, loss, esum = pl.pallas_call(
        _vq_body,
        grid=(_GRID,),
        in_specs=[
            pl.BlockSpec((_IMGS_PER_STEP, _EMBEDDING_DIM, _SPATIAL),
                         lambda i: (i, 0, 0)),
            pl.BlockSpec((_NUM_EMBEDDINGS, _EMBEDDING_DIM), lambda i: (0, 0)),
        ],
        out_specs=[
            pl.BlockSpec((_IMGS_PER_STEP, _EMBEDDING_DIM, _SPATIAL),
                         lambda i: (i, 0, 0)),
            pl.BlockSpec((_IMGS_PER_STEP, 1, _SPATIAL), lambda i: (i, 0, 0)),
            pl.BlockSpec((1, 1), lambda i: (0, 0)),
            pl.BlockSpec((1, 256), lambda i: (0, 0)),
        ],
        out_shape=[
            jax.ShapeDtypeStruct((_BATCH, _EMBEDDING_DIM, _SPATIAL), jnp.float32),
            jax.ShapeDtypeStruct((_BATCH, 1, _SPATIAL), jnp.int32),
            jax.ShapeDtypeStruct((1, 1), jnp.float32),
            jax.ShapeDtypeStruct((1, 256), jnp.float32),
        ],
        compiler_params=pltpu.CompilerParams(
            dimension_semantics=("arbitrary",),
        ),
    )(x, W)

    quantized_st = q.reshape(_BATCH, _EMBEDDING_DIM, 32, 32)
    encoding_indices = idx.reshape(_BATCH, 32, 32)
    return (quantized_st, loss[0, 0], encoding_indices, esum[0], W)
